# baseline (speedup 1.0000x reference)
import functools

import jax
import jax.numpy as jnp
from jax import lax
from jax.experimental import pallas as pl
from jax.experimental.pallas import tpu as pltpu
from jax.experimental.pallas import tpu_sc as plsc

K_POOL = 1024
N_CTX_ = 16
CTX_DIM_ = 768
TOPK = 32
L = 16
NC, NS = 1, 16
NVREG = CTX_DIM_ // L
CAND = 128


def _merge_split(ak, ai, bk, bi):
    rbk = lax.rev(bk, (0,))
    rbi = lax.rev(bi, (0,))
    take_a = ak >= rbk
    hk = jnp.where(take_a, ak, rbk)
    hi = jnp.where(take_a, ai, rbi)
    lk = jnp.where(take_a, rbk, ak)
    li = jnp.where(take_a, rbi, ai)
    hk, hi = plsc.sort_key_val(hk, hi, descending=True)
    lk, li = plsc.sort_key_val(lk, li, descending=True)
    return hk, hi, lk, li


def _top16_of(ak, ai, bk, bi):
    rbk = lax.rev(bk, (0,))
    rbi = lax.rev(bi, (0,))
    take_a = ak >= rbk
    hk = jnp.where(take_a, ak, rbk)
    hi = jnp.where(take_a, ai, rbi)
    return plsc.sort_key_val(hk, hi, descending=True)


def _sc_body(weights_hbm, table_hbm, out_hbm, w_v, idx_v, cand_k, cand_i,
             rows_v, acc_v, sem):
    wid = lax.axis_index("s") * NC + lax.axis_index("c")

    @pl.when(wid < N_CTX_)
    def _():
        n_row = wid

        pltpu.sync_copy(weights_hbm, w_v)

        i0 = lax.iota(jnp.int32, L)
        neg = jnp.full((L,), -3.0e38, jnp.float32)

        def _seed_step(j, c):
            m1, i1, m2, i2 = c
            off = pl.multiple_of(j * L, L)
            v = w_v[pl.ds(off, L)]
            iv = i0 + j * L
            gt1 = v > m1
            gt2 = v > m2
            m2n = jnp.where(gt1, m1, jnp.where(gt2, v, m2))
            i2n = jnp.where(gt1, i1, jnp.where(gt2, iv, i2))
            return (jnp.where(gt1, v, m1), jnp.where(gt1, iv, i1), m2n, i2n)

        def seed_body(q, c):
            for u in range(4):
                c = _seed_step(q * 4 + u, c)
            return c

        m1, i1, m2, i2 = lax.fori_loop(
            0, K_POOL // (4 * L), seed_body, (neg, i0, neg, i0))
        m1, i1 = plsc.sort_key_val(m1, i1, descending=True)
        m2, i2 = plsc.sort_key_val(m2, i2, descending=True)
        _, _, s1k, _ = _merge_split(m1, i1, m2, i2)
        thr = s1k[L - 1]

        for b in range(CAND // L):
            cand_k[pl.ds(b * L, L)] = neg

        def _ext_step(j, cnt):
            off = pl.multiple_of(j * L, L)
            v = w_v[pl.ds(off, L)]
            iv = i0 + j * L
            msk = v >= thr
            csum = plsc.cumsum(msk.astype(jnp.int32))
            pos = jnp.minimum(cnt + csum - 1, CAND - 1)
            plsc.store_scatter(cand_k, [pos], v, mask=msk)
            plsc.store_scatter(cand_i, [pos], iv, mask=msk)
            return cnt + csum[L - 1]

        def ext_body(q, cnt):
            for u in range(2):
                cnt = _ext_step(q * 2 + u, cnt)
            return cnt

        n_cand = lax.fori_loop(0, K_POOL // (2 * L), ext_body, jnp.int32(0))

        def _stream_topk(src_k, src_i, nvec):
            ak, ai = plsc.sort_key_val(src_k(0), src_i(0), descending=True)
            bk, bi = plsc.sort_key_val(src_k(1), src_i(1), descending=True)
            t0k, t0i, t1k, t1i = _merge_split(ak, ai, bk, bi)

            def body(j, carry):
                t0k, t0i, t1k, t1i, cthr = carry
                v = src_k(j)

                def do_merge(_):
                    vk, vi = plsc.sort_key_val(v, src_i(j), descending=True)
                    hk, hi = _top16_of(t1k, t1i, vk, vi)
                    n0k, n0i, n1k, n1i = _merge_split(t0k, t0i, hk, hi)
                    return n0k, n0i, n1k, n1i, n1k[L - 1]

                def skip(_):
                    return carry

                return lax.cond(jnp.max(v) > cthr, do_merge, skip, 0)

            t0k, t0i, t1k, t1i, _ = lax.fori_loop(
                2, nvec, body, (t0k, t0i, t1k, t1i, t1k[L - 1]))
            return t0k, t0i, t1k, t1i

        def fast_path(_):
            return _stream_topk(
                lambda j: cand_k[pl.ds(pl.multiple_of(j * L, L), L)],
                lambda j: cand_i[pl.ds(pl.multiple_of(j * L, L), L)],
                CAND // L)

        def slow_path(_):
            return _stream_topk(
                lambda j: w_v[pl.ds(pl.multiple_of(j * L, L), L)],
                lambda j: i0 + j * L,
                K_POOL // L)

        t0k, t0i, t1k, t1i = lax.cond(n_cand < CAND, fast_path, slow_path, 0)

        s = jnp.broadcast_to(jnp.sum(t0k) + jnp.sum(t1k), (L,))
        one = jnp.ones((L,), jnp.float32)
        r = one / s
        r = r * (2.0 - s * r)
        wn0 = t0k * r
        wn1 = t1k * r
        idx_v[pl.ds(0, L)] = t0i * N_CTX_ + n_row
        idx_v[pl.ds(L, L)] = t1i * N_CTX_ + n_row

        pltpu.async_copy(table_hbm.at[idx_v], rows_v, sem).wait()

        gdn = lax.GatherDimensionNumbers(
            offset_dims=(), collapsed_slice_dims=(0,), start_index_map=(0,))

        def _gather1d(src, lv):
            return lax.gather(
                src, lv[:, None], dimension_numbers=gdn, slice_sizes=(1,),
                mode=lax.GatherScatterMode.PROMISE_IN_BOUNDS)

        def _acc_step(i, acc):
            iv = jnp.broadcast_to(i, (L,)).astype(jnp.int32)
            lv = jnp.bitwise_and(iv, L - 1)
            wv = jnp.where(iv < L, _gather1d(wn0, lv), _gather1d(wn1, lv))
            return tuple(
                acc[c] + rows_v[i, pl.ds(c * L, L)] * wv
                for c in range(NVREG))

        def acc_body(q, acc):
            return _acc_step(2 * q + 1, _acc_step(2 * q, acc))

        acc = lax.fori_loop(
            0, TOPK // 2, acc_body,
            tuple(jnp.zeros((L,), jnp.float32) for _ in range(NVREG)))
        for c in range(NVREG):
            acc_v[pl.ds(c * L, L)] = acc[c]

        pltpu.sync_copy(acc_v, out_hbm.at[n_row])


_mesh = plsc.VectorSubcoreMesh(core_axis_name="c", subcore_axis_name="s",
                               num_cores=NC, num_subcores=NS)

_prompt_pool_sc = functools.partial(
    pl.kernel,
    out_type=jax.ShapeDtypeStruct((N_CTX_, CTX_DIM_), jnp.float32),
    mesh=_mesh,
    scratch_types=[
        pltpu.VMEM((K_POOL,), jnp.float32),
        pltpu.VMEM((TOPK,), jnp.int32),
        pltpu.VMEM((CAND,), jnp.float32),
        pltpu.VMEM((CAND,), jnp.int32),
        pltpu.VMEM((TOPK, CTX_DIM_), jnp.float32),
        pltpu.VMEM((CTX_DIM_,), jnp.float32),
        pltpu.SemaphoreType.DMA,
    ],
    compiler_params=pltpu.CompilerParams(needs_layout_passes=False),
)(_sc_body)


def kernel(weights, prompts, top_m):
    del top_m
    table = prompts.reshape(K_POOL * N_CTX_, CTX_DIM_)
    return _prompt_pool_sc(weights, table)


# speedup vs baseline: 1.0396x; 1.0396x over previous
import functools

import jax
import jax.numpy as jnp
from jax import lax
from jax.experimental import pallas as pl
from jax.experimental.pallas import tpu as pltpu
from jax.experimental.pallas import tpu_sc as plsc

K_POOL = 1024
N_CTX_ = 16
CTX_DIM_ = 768
TOPK = 32
L = 16
NC, NS = 1, 16
NVREG = CTX_DIM_ // L
CAND = 128


def _merge_split(ak, ai, bk, bi):
    rbk = lax.rev(bk, (0,))
    rbi = lax.rev(bi, (0,))
    take_a = ak >= rbk
    hk = jnp.where(take_a, ak, rbk)
    hi = jnp.where(take_a, ai, rbi)
    lk = jnp.where(take_a, rbk, ak)
    li = jnp.where(take_a, rbi, ai)
    hk, hi = plsc.sort_key_val(hk, hi, descending=True)
    lk, li = plsc.sort_key_val(lk, li, descending=True)
    return hk, hi, lk, li


def _top16_of(ak, ai, bk, bi):
    rbk = lax.rev(bk, (0,))
    rbi = lax.rev(bi, (0,))
    take_a = ak >= rbk
    hk = jnp.where(take_a, ak, rbk)
    hi = jnp.where(take_a, ai, rbi)
    return plsc.sort_key_val(hk, hi, descending=True)


def _sc_body(weights_hbm, table_hbm, out_hbm, w_v, idx_v, cand_k, cand_i,
             rows_v, acc_v, sem):
    if True:
        n_row = lax.axis_index("s")

        pltpu.sync_copy(weights_hbm, w_v)

        i0 = lax.iota(jnp.int32, L)
        neg = jnp.full((L,), -3.0e38, jnp.float32)

        def seed_body(j, c):
            m1, i1, m2, i2 = c
            off = pl.multiple_of(j * L, L)
            v = w_v[pl.ds(off, L)]
            iv = i0 + j * L
            gt1 = v > m1
            gt2 = v > m2
            m2n = jnp.where(gt1, m1, jnp.where(gt2, v, m2))
            i2n = jnp.where(gt1, i1, jnp.where(gt2, iv, i2))
            return (jnp.where(gt1, v, m1), jnp.where(gt1, iv, i1), m2n, i2n)

        m1, i1, m2, i2 = lax.fori_loop(
            0, K_POOL // L, seed_body, (neg, i0, neg, i0))
        m1, i1 = plsc.sort_key_val(m1, i1, descending=True)
        m2, i2 = plsc.sort_key_val(m2, i2, descending=True)
        _, _, s1k, _ = _merge_split(m1, i1, m2, i2)
        thr = s1k[L - 1]

        for b in range(CAND // L):
            cand_k[pl.ds(b * L, L)] = neg

        def ext_body(j, cnt):
            off = pl.multiple_of(j * L, L)
            v = w_v[pl.ds(off, L)]
            iv = i0 + j * L
            msk = v >= thr
            csum = plsc.cumsum(msk.astype(jnp.int32))
            pos = jnp.minimum(cnt + csum - 1, CAND - 1)
            plsc.store_scatter(cand_k, [pos], v, mask=msk)
            plsc.store_scatter(cand_i, [pos], iv, mask=msk)
            return cnt + csum[L - 1]

        n_cand = lax.fori_loop(0, K_POOL // L, ext_body, jnp.int32(0))

        def _stream_topk(src_k, src_i, nvec):
            ak, ai = plsc.sort_key_val(src_k(0), src_i(0), descending=True)
            bk, bi = plsc.sort_key_val(src_k(1), src_i(1), descending=True)
            t0k, t0i, t1k, t1i = _merge_split(ak, ai, bk, bi)

            def body(j, carry):
                t0k, t0i, t1k, t1i, cthr = carry
                v = src_k(j)

                def do_merge(_):
                    vk, vi = plsc.sort_key_val(v, src_i(j), descending=True)
                    hk, hi = _top16_of(t1k, t1i, vk, vi)
                    n0k, n0i, n1k, n1i = _merge_split(t0k, t0i, hk, hi)
                    return n0k, n0i, n1k, n1i, n1k[L - 1]

                def skip(_):
                    return carry

                return lax.cond(jnp.max(v) > cthr, do_merge, skip, 0)

            t0k, t0i, t1k, t1i, _ = lax.fori_loop(
                2, nvec, body, (t0k, t0i, t1k, t1i, t1k[L - 1]))
            return t0k, t0i, t1k, t1i

        def fast_path(_):
            return _stream_topk(
                lambda j: cand_k[pl.ds(pl.multiple_of(j * L, L), L)],
                lambda j: cand_i[pl.ds(pl.multiple_of(j * L, L), L)],
                CAND // L)

        def slow_path(_):
            return _stream_topk(
                lambda j: w_v[pl.ds(pl.multiple_of(j * L, L), L)],
                lambda j: i0 + j * L,
                K_POOL // L)

        t0k, t0i, t1k, t1i = lax.cond(n_cand < CAND, fast_path, slow_path, 0)

        s = jnp.broadcast_to(jnp.sum(t0k) + jnp.sum(t1k), (L,))
        one = jnp.ones((L,), jnp.float32)
        r = one / s
        r = r * (2.0 - s * r)
        wn0 = t0k * r
        wn1 = t1k * r
        idx_v[pl.ds(0, L)] = t0i * N_CTX_ + n_row
        idx_v[pl.ds(L, L)] = t1i * N_CTX_ + n_row

        pltpu.async_copy(table_hbm.at[idx_v], rows_v, sem).wait()

        gdn = lax.GatherDimensionNumbers(
            offset_dims=(), collapsed_slice_dims=(0,), start_index_map=(0,))

        def _gather1d(src, lv):
            return lax.gather(
                src, lv[:, None], dimension_numbers=gdn, slice_sizes=(1,),
                mode=lax.GatherScatterMode.PROMISE_IN_BOUNDS)

        def acc_body(i, acc):
            iv = jnp.broadcast_to(i, (L,)).astype(jnp.int32)
            lv = jnp.bitwise_and(iv, L - 1)
            wv = jnp.where(iv < L, _gather1d(wn0, lv), _gather1d(wn1, lv))
            return tuple(
                acc[c] + rows_v[i, pl.ds(c * L, L)] * wv
                for c in range(NVREG))

        acc = lax.fori_loop(
            0, TOPK, acc_body,
            tuple(jnp.zeros((L,), jnp.float32) for _ in range(NVREG)))
        for c in range(NVREG):
            acc_v[pl.ds(c * L, L)] = acc[c]

        pltpu.sync_copy(acc_v, out_hbm.at[n_row])


_mesh = plsc.VectorSubcoreMesh(core_axis_name="c", subcore_axis_name="s",
                               num_cores=NC, num_subcores=NS)

_prompt_pool_sc = functools.partial(
    pl.kernel,
    out_type=jax.ShapeDtypeStruct((N_CTX_, CTX_DIM_), jnp.float32),
    mesh=_mesh,
    scratch_types=[
        pltpu.VMEM((K_POOL,), jnp.float32),
        pltpu.VMEM((TOPK,), jnp.int32),
        pltpu.VMEM((CAND,), jnp.float32),
        pltpu.VMEM((CAND,), jnp.int32),
        pltpu.VMEM((TOPK, CTX_DIM_), jnp.float32),
        pltpu.VMEM((CTX_DIM_,), jnp.float32),
        pltpu.SemaphoreType.DMA,
    ],
    compiler_params=pltpu.CompilerParams(needs_layout_passes=False),
)(_sc_body)


def kernel(weights, prompts, top_m):
    del top_m
    table = prompts.reshape(K_POOL * N_CTX_, CTX_DIM_)
    return _prompt_pool_sc(weights, table)


# speedup vs baseline: 1.0405x; 1.0008x over previous
import functools

import jax
import jax.numpy as jnp
from jax import lax
from jax.experimental import pallas as pl
from jax.experimental.pallas import tpu as pltpu
from jax.experimental.pallas import tpu_sc as plsc

K_POOL = 1024
N_CTX_ = 16
CTX_DIM_ = 768
TOPK = 32
L = 16
NC, NS = 1, 16
NVREG = CTX_DIM_ // L
CAND = 128


def _merge_split(ak, ai, bk, bi):
    rbk = lax.rev(bk, (0,))
    rbi = lax.rev(bi, (0,))
    take_a = ak >= rbk
    hk = jnp.where(take_a, ak, rbk)
    hi = jnp.where(take_a, ai, rbi)
    lk = jnp.where(take_a, rbk, ak)
    li = jnp.where(take_a, rbi, ai)
    hk, hi = plsc.sort_key_val(hk, hi, descending=True)
    lk, li = plsc.sort_key_val(lk, li, descending=True)
    return hk, hi, lk, li


def _top16_of(ak, ai, bk, bi):
    rbk = lax.rev(bk, (0,))
    rbi = lax.rev(bi, (0,))
    take_a = ak >= rbk
    hk = jnp.where(take_a, ak, rbk)
    hi = jnp.where(take_a, ai, rbi)
    return plsc.sort_key_val(hk, hi, descending=True)


def _sc_body(weights_hbm, table_hbm, out_hbm, w_v, idx_v, cand_k, cand_i,
             rows_v, acc_v, sem):
    n_row = lax.axis_index("s")

    pltpu.sync_copy(weights_hbm, w_v)

    i0 = lax.iota(jnp.int32, L)
    neg = jnp.full((L,), -3.0e38, jnp.float32)

    def seed_body(j, c):
        m1, i1, m2, i2 = c
        off = pl.multiple_of(j * L, L)
        v = w_v[pl.ds(off, L)]
        iv = i0 + j * L
        gt1 = v > m1
        gt2 = v > m2
        m2n = jnp.where(gt1, m1, jnp.where(gt2, v, m2))
        i2n = jnp.where(gt1, i1, jnp.where(gt2, iv, i2))
        return (jnp.where(gt1, v, m1), jnp.where(gt1, iv, i1), m2n, i2n)

    m1, i1, m2, i2 = lax.fori_loop(
        0, K_POOL // L, seed_body, (neg, i0, neg, i0))
    m1, i1 = plsc.sort_key_val(m1, i1, descending=True)
    m2, i2 = plsc.sort_key_val(m2, i2, descending=True)
    _, _, s1k, _ = _merge_split(m1, i1, m2, i2)
    thr = s1k[L - 1]

    for b in range(CAND // L):
        cand_k[pl.ds(b * L, L)] = neg

    def ext_body(j, cnt):
        off = pl.multiple_of(j * L, L)
        v = w_v[pl.ds(off, L)]
        iv = i0 + j * L
        msk = v >= thr
        csum = plsc.cumsum(msk.astype(jnp.int32))
        pos = jnp.minimum(cnt + csum - 1, CAND - 1)
        plsc.store_scatter(cand_k, [pos], v, mask=msk)
        plsc.store_scatter(cand_i, [pos], iv, mask=msk)
        return cnt + csum[L - 1]

    n_cand = lax.fori_loop(0, K_POOL // L, ext_body, jnp.int32(0))

    def _stream_topk(src_k, src_i, nvec):
        ak, ai = plsc.sort_key_val(src_k(0), src_i(0), descending=True)
        bk, bi = plsc.sort_key_val(src_k(1), src_i(1), descending=True)
        t0k, t0i, t1k, t1i = _merge_split(ak, ai, bk, bi)

        def body(j, carry):
            t0k, t0i, t1k, t1i, cthr = carry
            v = src_k(j)

            def do_merge(_):
                vk, vi = plsc.sort_key_val(v, src_i(j), descending=True)
                hk, hi = _top16_of(t1k, t1i, vk, vi)
                n0k, n0i, n1k, n1i = _merge_split(t0k, t0i, hk, hi)
                return n0k, n0i, n1k, n1i, n1k[L - 1]

            def skip(_):
                return carry

            return lax.cond(jnp.max(v) > cthr, do_merge, skip, 0)

        t0k, t0i, t1k, t1i, _ = lax.fori_loop(
            2, nvec, body, (t0k, t0i, t1k, t1i, t1k[L - 1]))
        return t0k, t0i, t1k, t1i

    def fast_path(_):
        return _stream_topk(
            lambda j: cand_k[pl.ds(pl.multiple_of(j * L, L), L)],
            lambda j: cand_i[pl.ds(pl.multiple_of(j * L, L), L)],
            CAND // L)

    def slow_path(_):
        return _stream_topk(
            lambda j: w_v[pl.ds(pl.multiple_of(j * L, L), L)],
            lambda j: i0 + j * L,
            K_POOL // L)

    t0k, t0i, t1k, t1i = lax.cond(n_cand < CAND, fast_path, slow_path, 0)

    s = jnp.broadcast_to(jnp.sum(t0k) + jnp.sum(t1k), (L,))
    one = jnp.ones((L,), jnp.float32)
    r = one / s
    r = r * (2.0 - s * r)
    wn0 = t0k * r
    wn1 = t1k * r
    idx_v[pl.ds(0, L)] = t0i * N_CTX_ + n_row
    idx_v[pl.ds(L, L)] = t1i * N_CTX_ + n_row

    pltpu.async_copy(table_hbm.at[idx_v], rows_v, sem).wait()

    gdn = lax.GatherDimensionNumbers(
        offset_dims=(), collapsed_slice_dims=(0,), start_index_map=(0,))

    def _gather1d(src, lv):
        return lax.gather(
            src, lv[:, None], dimension_numbers=gdn, slice_sizes=(1,),
            mode=lax.GatherScatterMode.PROMISE_IN_BOUNDS)

    def acc_body(i, acc):
        iv = jnp.broadcast_to(i, (L,)).astype(jnp.int32)
        lv = jnp.bitwise_and(iv, L - 1)
        wv = jnp.where(iv < L, _gather1d(wn0, lv), _gather1d(wn1, lv))
        return tuple(
            acc[c] + rows_v[i, pl.ds(c * L, L)] * wv
            for c in range(NVREG))

    acc = lax.fori_loop(
        0, TOPK, acc_body,
        tuple(jnp.zeros((L,), jnp.float32) for _ in range(NVREG)))
    for c in range(NVREG):
        acc_v[pl.ds(c * L, L)] = acc[c]

    pltpu.sync_copy(acc_v, out_hbm.at[n_row])


_mesh = plsc.VectorSubcoreMesh(core_axis_name="c", subcore_axis_name="s",
                               num_cores=NC, num_subcores=NS)

_prompt_pool_sc = functools.partial(
    pl.kernel,
    out_type=jax.ShapeDtypeStruct((N_CTX_, CTX_DIM_), jnp.float32),
    mesh=_mesh,
    scratch_types=[
        pltpu.VMEM((K_POOL,), jnp.float32),
        pltpu.VMEM((TOPK,), jnp.int32),
        pltpu.VMEM((CAND,), jnp.float32),
        pltpu.VMEM((CAND,), jnp.int32),
        pltpu.VMEM((TOPK, CTX_DIM_), jnp.float32),
        pltpu.VMEM((CTX_DIM_,), jnp.float32),
        pltpu.SemaphoreType.DMA,
    ],
    compiler_params=pltpu.CompilerParams(needs_layout_passes=False),
)(_sc_body)


def kernel(weights, prompts, top_m):
    del top_m
    table = prompts.reshape(K_POOL * N_CTX_, CTX_DIM_)
    return _prompt_pool_sc(weights, table)
